# baseline (speedup 1.0000x reference)
import functools
import numpy as np
import jax
import jax.numpy as jnp
from jax import lax
from jax.experimental import pallas as pl
from jax.experimental.pallas import tpu as pltpu
from jax.experimental.pallas import tpu_sc as plsc

NUM_CLASSES = 100
LAB_PAD = 128
NUM_EMB = 50000
EMB_DIM = 768
OUT_DIM = 3072
BATCH = 8192

NC, NS = 2, 16
NW = NC * NS
ROWS_PW = BATCH // NW
CHUNK = 64
NCHUNK = ROWS_PW // CHUNK


def _build_upsample_matrix() -> np.ndarray:
    U = np.zeros((32, 16), np.float32)
    for i in range(32):
        c = (i + 0.5) / 2.0 - 0.5
        f = int(np.floor(c))
        t = c - f
        for (j, w) in ((f, 1.0 - t), (f + 1, t)):
            U[i, min(max(j, 0), 15)] += w
    K = np.einsum("ih,jw->hwij", U, U).reshape(256, 1024)
    M = np.zeros((EMB_DIM, OUT_DIM), np.float32)
    for c in range(3):
        M[c * 256:(c + 1) * 256, c * 1024:(c + 1) * 1024] = K
    return M


def _build_upsample_blocks() -> np.ndarray:
    M = _build_upsample_matrix()
    K = M[:256, :1024]
    return np.ascontiguousarray(K.T)


_KT_NP = _build_upsample_blocks().astype(jnp.bfloat16)

_EYE_NP = np.eye(NUM_CLASSES, LAB_PAD, dtype=np.float32)

_sc_mesh = plsc.VectorSubcoreMesh(core_axis_name="c", subcore_axis_name="s")


@functools.partial(
    pl.kernel,
    mesh=_sc_mesh,
    out_type=jax.ShapeDtypeStruct((BATCH, EMB_DIM), jnp.float32),
    scratch_types=[
        [pltpu.VMEM((CHUNK,), jnp.int32) for _ in range(NCHUNK)],
        pltpu.VMEM((CHUNK, EMB_DIM), jnp.float32),
        pltpu.VMEM((CHUNK, EMB_DIM), jnp.float32),
        pltpu.SemaphoreType.DMA,
        pltpu.SemaphoreType.DMA,
    ],
)
def _sc_gather_data(idx_hbm, data_hbm, outd_hbm, idx_bufs, rows_a, rows_b,
                    sem_a, sem_b):
    wid = lax.axis_index("s") * NC + lax.axis_index("c")
    base = wid * ROWS_PW
    for j in range(NCHUNK):
        pltpu.sync_copy(idx_hbm.at[pl.ds(base + j * CHUNK, CHUNK)], idx_bufs[j])
    bufs = [(rows_a, sem_a), (rows_b, sem_b)]
    dmas = [None, None]
    dmas[0] = pltpu.async_copy(data_hbm.at[idx_bufs[0]], rows_a, sem_a)
    for j in range(NCHUNK):
        buf, _ = bufs[j % 2]
        dmas[j % 2].wait()
        if j + 1 < NCHUNK:
            nbuf, nsem = bufs[(j + 1) % 2]
            dmas[(j + 1) % 2] = pltpu.async_copy(
                data_hbm.at[idx_bufs[j + 1]], nbuf, nsem)
        pltpu.sync_copy(buf, outd_hbm.at[pl.ds(base + j * CHUNK, CHUNK)])


@functools.partial(
    pl.kernel,
    mesh=_sc_mesh,
    out_type=jax.ShapeDtypeStruct((BATCH, LAB_PAD), jnp.float32),
    scratch_types=[
        pltpu.VMEM((ROWS_PW,), jnp.int32),
        pltpu.VMEM((ROWS_PW, LAB_PAD), jnp.float32),
        pltpu.SemaphoreType.DMA,
    ],
)
def _sc_gather_labels(idx_hbm, labp_hbm, outl_hbm, idx_v, lab_v, sem_l):
    wid = lax.axis_index("s") * NC + lax.axis_index("c")
    base = wid * ROWS_PW
    pltpu.sync_copy(idx_hbm.at[pl.ds(base, ROWS_PW)], idx_v)
    pltpu.async_copy(labp_hbm.at[idx_v], lab_v, sem_l).wait()
    pltpu.sync_copy(lab_v, outl_hbm.at[pl.ds(base, ROWS_PW)])


def _tc_slice_body(x_ref, eye_ref, o_ref):
    o_ref[...] = lax.dot_general(
        eye_ref[...], x_ref[...], (((1,), (1,)), ((), ())),
        preferred_element_type=jnp.float32)


_SLC_BLK = 2048


def _tc_slice_labels(labp_rows):
    return pl.pallas_call(
        _tc_slice_body,
        grid=(BATCH // _SLC_BLK,),
        in_specs=[
            pl.BlockSpec((_SLC_BLK, LAB_PAD), lambda i: (i, 0)),
            pl.BlockSpec((NUM_CLASSES, LAB_PAD), lambda i: (0, 0)),
        ],
        out_specs=pl.BlockSpec((NUM_CLASSES, _SLC_BLK), lambda i: (0, i)),
        out_shape=jax.ShapeDtypeStruct((NUM_CLASSES, BATCH), jnp.float32),
    )(labp_rows, jnp.asarray(_EYE_NP))


def _tc_padT_body(xt_ref, eye_ref, o_ref):
    o_ref[...] = lax.dot_general(
        xt_ref[...], eye_ref[...], (((0,), (0,)), ((), ())),
        preferred_element_type=jnp.float32)


_PAD_BLK = 2048


def _tc_pad_labels(label_table_t):
    return pl.pallas_call(
        _tc_padT_body,
        grid=(pl.cdiv(NUM_EMB, _PAD_BLK),),
        in_specs=[
            pl.BlockSpec((NUM_CLASSES, _PAD_BLK), lambda i: (0, i)),
            pl.BlockSpec((NUM_CLASSES, LAB_PAD), lambda i: (0, 0)),
        ],
        out_specs=pl.BlockSpec((_PAD_BLK, LAB_PAD), lambda i: (i, 0)),
        out_shape=jax.ShapeDtypeStruct((NUM_EMB, LAB_PAD), jnp.float32),
    )(label_table_t, jnp.asarray(_EYE_NP))


def _tc_upsample_body(x_ref, kt_ref, o_ref):
    x = x_ref[...].astype(jnp.bfloat16)
    o_ref[...] = lax.dot_general(
        kt_ref[...], x, (((1,), (1,)), ((), ())),
        preferred_element_type=jnp.float32)


_TC_BLK = 512
_CH_IN = 256
_CH_OUT = 1024


def _tc_upsample(gathered):
    return pl.pallas_call(
        _tc_upsample_body,
        grid=(3, BATCH // _TC_BLK),
        in_specs=[
            pl.BlockSpec((_TC_BLK, _CH_IN), lambda c, i: (i, c)),
            pl.BlockSpec((_CH_OUT, _CH_IN), lambda c, i: (0, 0)),
        ],
        out_specs=pl.BlockSpec((_CH_OUT, _TC_BLK), lambda c, i: (c, i)),
        out_shape=jax.ShapeDtypeStruct((OUT_DIM, BATCH), jnp.float32),
    )(gathered, jnp.asarray(_KT_NP))


@jax.jit
def kernel(indices, data_table, label_table):
    labp = _tc_pad_labels(label_table.T)
    gathered = _sc_gather_data(indices, data_table)
    labels_t = _tc_slice_labels(_sc_gather_labels(indices, labp))
    imgs_t = _tc_upsample(gathered)
    imgs = imgs_t.reshape(3, 32, 32, BATCH).transpose(3, 0, 1, 2)
    return imgs, labels_t.T


# speedup vs baseline: 1.2695x; 1.2695x over previous
import functools
import numpy as np
import jax
import jax.numpy as jnp
from jax import lax
from jax.experimental import pallas as pl
from jax.experimental.pallas import tpu as pltpu
from jax.experimental.pallas import tpu_sc as plsc

NUM_CLASSES = 100
LAB_PAD = 128
NUM_EMB = 50000
EMB_DIM = 768
OUT_DIM = 3072
BATCH = 8192

NC, NS = 2, 16
NW = NC * NS
ROWS_PW = BATCH // NW
CHUNK = 64
NCHUNK = ROWS_PW // CHUNK


def _build_upsample_matrix() -> np.ndarray:
    U = np.zeros((32, 16), np.float32)
    for i in range(32):
        c = (i + 0.5) / 2.0 - 0.5
        f = int(np.floor(c))
        t = c - f
        for (j, w) in ((f, 1.0 - t), (f + 1, t)):
            U[i, min(max(j, 0), 15)] += w
    K = np.einsum("ih,jw->hwij", U, U).reshape(256, 1024)
    M = np.zeros((EMB_DIM, OUT_DIM), np.float32)
    for c in range(3):
        M[c * 256:(c + 1) * 256, c * 1024:(c + 1) * 1024] = K
    return M


def _build_upsample_blocks() -> np.ndarray:
    M = _build_upsample_matrix()
    K = M[:256, :1024]
    return np.ascontiguousarray(K.T)


_KT_NP = _build_upsample_blocks().astype(jnp.bfloat16)

_EYE_NP = np.eye(NUM_CLASSES, LAB_PAD, dtype=np.float32)

_sc_mesh = plsc.VectorSubcoreMesh(core_axis_name="c", subcore_axis_name="s")


@functools.partial(
    pl.kernel,
    mesh=_sc_mesh,
    out_type=jax.ShapeDtypeStruct((BATCH, EMB_DIM), jnp.float32),
    scratch_types=[
        [pltpu.VMEM((CHUNK,), jnp.int32) for _ in range(NCHUNK)],
        pltpu.VMEM((CHUNK, EMB_DIM), jnp.float32),
        pltpu.VMEM((CHUNK, EMB_DIM), jnp.float32),
        pltpu.SemaphoreType.DMA,
        pltpu.SemaphoreType.DMA,
    ],
)
def _sc_gather_data(idx_hbm, data_hbm, outd_hbm, idx_bufs, rows_a, rows_b,
                    sem_a, sem_b):
    wid = lax.axis_index("s") * NC + lax.axis_index("c")
    base = wid * ROWS_PW
    for j in range(NCHUNK):
        pltpu.sync_copy(idx_hbm.at[pl.ds(base + j * CHUNK, CHUNK)], idx_bufs[j])
    bufs = [(rows_a, sem_a), (rows_b, sem_b)]
    dmas = [None, None]
    dmas[0] = pltpu.async_copy(data_hbm.at[idx_bufs[0]], rows_a, sem_a)
    for j in range(NCHUNK):
        buf, _ = bufs[j % 2]
        dmas[j % 2].wait()
        if j + 1 < NCHUNK:
            nbuf, nsem = bufs[(j + 1) % 2]
            dmas[(j + 1) % 2] = pltpu.async_copy(
                data_hbm.at[idx_bufs[j + 1]], nbuf, nsem)
        pltpu.sync_copy(buf, outd_hbm.at[pl.ds(base + j * CHUNK, CHUNK)])


@functools.partial(
    pl.kernel,
    mesh=_sc_mesh,
    out_type=jax.ShapeDtypeStruct((BATCH, LAB_PAD), jnp.float32),
    scratch_types=[
        pltpu.VMEM((ROWS_PW,), jnp.int32),
        pltpu.VMEM((ROWS_PW, LAB_PAD), jnp.float32),
        pltpu.SemaphoreType.DMA,
    ],
)
def _sc_gather_labels(idx_hbm, labp_hbm, outl_hbm, idx_v, lab_v, sem_l):
    wid = lax.axis_index("s") * NC + lax.axis_index("c")
    base = wid * ROWS_PW
    pltpu.sync_copy(idx_hbm.at[pl.ds(base, ROWS_PW)], idx_v)
    pltpu.async_copy(labp_hbm.at[idx_v], lab_v, sem_l).wait()
    pltpu.sync_copy(lab_v, outl_hbm.at[pl.ds(base, ROWS_PW)])


def _tc_slice_body(x_ref, eye_ref, o_ref):
    o_ref[...] = lax.dot_general(
        eye_ref[...], x_ref[...], (((1,), (1,)), ((), ())),
        preferred_element_type=jnp.float32)


_SLC_BLK = 2048


def _tc_slice_labels(labp_rows):
    return pl.pallas_call(
        _tc_slice_body,
        grid=(BATCH // _SLC_BLK,),
        in_specs=[
            pl.BlockSpec((_SLC_BLK, LAB_PAD), lambda i: (i, 0)),
            pl.BlockSpec((NUM_CLASSES, LAB_PAD), lambda i: (0, 0)),
        ],
        out_specs=pl.BlockSpec((NUM_CLASSES, _SLC_BLK), lambda i: (0, i)),
        out_shape=jax.ShapeDtypeStruct((NUM_CLASSES, BATCH), jnp.float32),
    )(labp_rows, jnp.asarray(_EYE_NP))


def _tc_padT_body(xt_ref, eye_ref, o_ref):
    o_ref[...] = lax.dot_general(
        xt_ref[...], eye_ref[...], (((0,), (0,)), ((), ())),
        preferred_element_type=jnp.float32)


_PAD_BLK = 8192


def _tc_pad_labels(label_table_t):
    return pl.pallas_call(
        _tc_padT_body,
        grid=(pl.cdiv(NUM_EMB, _PAD_BLK),),
        in_specs=[
            pl.BlockSpec((NUM_CLASSES, _PAD_BLK), lambda i: (0, i)),
            pl.BlockSpec((NUM_CLASSES, LAB_PAD), lambda i: (0, 0)),
        ],
        out_specs=pl.BlockSpec((_PAD_BLK, LAB_PAD), lambda i: (i, 0)),
        out_shape=jax.ShapeDtypeStruct((NUM_EMB, LAB_PAD), jnp.float32),
    )(label_table_t, jnp.asarray(_EYE_NP))


def _tc_upsample_body(x_ref, kt_ref, o_ref):
    x = x_ref[...].astype(jnp.bfloat16)
    o_ref[...] = lax.dot_general(
        kt_ref[...], x, (((1,), (1,)), ((), ())),
        preferred_element_type=jnp.float32)


_TC_BLK = 2048
_CH_IN = 256
_CH_OUT = 1024


def _tc_upsample(gathered):
    return pl.pallas_call(
        _tc_upsample_body,
        grid=(3, BATCH // _TC_BLK),
        in_specs=[
            pl.BlockSpec((_TC_BLK, _CH_IN), lambda c, i: (i, c)),
            pl.BlockSpec((_CH_OUT, _CH_IN), lambda c, i: (0, 0)),
        ],
        out_specs=pl.BlockSpec((_CH_OUT, _TC_BLK), lambda c, i: (c, i)),
        out_shape=jax.ShapeDtypeStruct((OUT_DIM, BATCH), jnp.float32),
    )(gathered, jnp.asarray(_KT_NP))


@jax.jit
def kernel(indices, data_table, label_table):
    gathered = _sc_gather_data(indices, data_table)
    labp = _tc_pad_labels(label_table.T)
    labels_t = _tc_slice_labels(_sc_gather_labels(indices, labp))
    imgs_t = _tc_upsample(gathered)
    imgs = imgs_t.reshape(3, 32, 32, BATCH).transpose(3, 0, 1, 2)
    return imgs, labels_t.T


# speedup vs baseline: 1.3687x; 1.0781x over previous
import functools
import numpy as np
import jax
import jax.numpy as jnp
from jax import lax
from jax.experimental import pallas as pl
from jax.experimental.pallas import tpu as pltpu
from jax.experimental.pallas import tpu_sc as plsc

NUM_CLASSES = 100
LAB_PAD = 128
NUM_EMB = 50000
EMB_DIM = 768
OUT_DIM = 3072
BATCH = 8192

NC, NS = 2, 16
NW = NC * NS
ROWS_PW = BATCH // NW
CHUNK = 64
NCHUNK = ROWS_PW // CHUNK


def _build_upsample_matrix() -> np.ndarray:
    U = np.zeros((32, 16), np.float32)
    for i in range(32):
        c = (i + 0.5) / 2.0 - 0.5
        f = int(np.floor(c))
        t = c - f
        for (j, w) in ((f, 1.0 - t), (f + 1, t)):
            U[i, min(max(j, 0), 15)] += w
    K = np.einsum("ih,jw->hwij", U, U).reshape(256, 1024)
    M = np.zeros((EMB_DIM, OUT_DIM), np.float32)
    for c in range(3):
        M[c * 256:(c + 1) * 256, c * 1024:(c + 1) * 1024] = K
    return M


def _build_upsample_blocks() -> np.ndarray:
    M = _build_upsample_matrix()
    K = M[:256, :1024]
    return np.ascontiguousarray(K.T)


_KT_NP = _build_upsample_blocks().astype(jnp.bfloat16)

_EYE_NP = np.eye(NUM_CLASSES, LAB_PAD, dtype=np.float32)

_sc_mesh = plsc.VectorSubcoreMesh(core_axis_name="c", subcore_axis_name="s")


@functools.partial(
    pl.kernel,
    mesh=_sc_mesh,
    out_type=jax.ShapeDtypeStruct((BATCH, EMB_DIM), jnp.float32),
    scratch_types=[
        pltpu.VMEM((ROWS_PW,), jnp.int32),
        pltpu.VMEM((CHUNK, EMB_DIM), jnp.float32),
        pltpu.VMEM((CHUNK, EMB_DIM), jnp.float32),
        pltpu.SemaphoreType.DMA,
        pltpu.SemaphoreType.DMA,
    ],
)
def _sc_gather_data(idx_hbm, data_hbm, outd_hbm, idx_v, rows_a, rows_b,
                    sem_a, sem_b):
    wid = lax.axis_index("s") * NC + lax.axis_index("c")
    base = wid * ROWS_PW
    pltpu.sync_copy(idx_hbm.at[pl.ds(base, ROWS_PW)], idx_v)
    bufs = [(rows_a, sem_a), (rows_b, sem_b)]
    dmas = [None, None]
    dmas[0] = pltpu.async_copy(
        data_hbm.at[idx_v.at[pl.ds(0, CHUNK)]], rows_a, sem_a)
    for j in range(NCHUNK):
        buf, _ = bufs[j % 2]
        dmas[j % 2].wait()
        if j + 1 < NCHUNK:
            nbuf, nsem = bufs[(j + 1) % 2]
            dmas[(j + 1) % 2] = pltpu.async_copy(
                data_hbm.at[idx_v.at[pl.ds((j + 1) * CHUNK, CHUNK)]],
                nbuf, nsem)
        pltpu.sync_copy(buf, outd_hbm.at[pl.ds(base + j * CHUNK, CHUNK)])


@functools.partial(
    pl.kernel,
    mesh=_sc_mesh,
    out_type=jax.ShapeDtypeStruct((BATCH, LAB_PAD), jnp.float32),
    scratch_types=[
        pltpu.VMEM((ROWS_PW,), jnp.int32),
        pltpu.VMEM((ROWS_PW, LAB_PAD), jnp.float32),
        pltpu.SemaphoreType.DMA,
    ],
)
def _sc_gather_labels(idx_hbm, labp_hbm, outl_hbm, idx_v, lab_v, sem_l):
    wid = lax.axis_index("s") * NC + lax.axis_index("c")
    base = wid * ROWS_PW
    pltpu.sync_copy(idx_hbm.at[pl.ds(base, ROWS_PW)], idx_v)
    pltpu.async_copy(labp_hbm.at[idx_v], lab_v, sem_l).wait()
    pltpu.sync_copy(lab_v, outl_hbm.at[pl.ds(base, ROWS_PW)])


def _tc_slice_body(x_ref, eye_ref, o_ref):
    o_ref[...] = lax.dot_general(
        eye_ref[...], x_ref[...], (((1,), (1,)), ((), ())),
        preferred_element_type=jnp.float32)


_SLC_BLK = 2048


def _tc_slice_labels(labp_rows):
    return pl.pallas_call(
        _tc_slice_body,
        grid=(BATCH // _SLC_BLK,),
        in_specs=[
            pl.BlockSpec((_SLC_BLK, LAB_PAD), lambda i: (i, 0)),
            pl.BlockSpec((NUM_CLASSES, LAB_PAD), lambda i: (0, 0)),
        ],
        out_specs=pl.BlockSpec((NUM_CLASSES, _SLC_BLK), lambda i: (0, i)),
        out_shape=jax.ShapeDtypeStruct((NUM_CLASSES, BATCH), jnp.float32),
    )(labp_rows, jnp.asarray(_EYE_NP))


def _tc_padT_body(xt_ref, eye_ref, o_ref):
    o_ref[...] = lax.dot_general(
        xt_ref[...], eye_ref[...], (((0,), (0,)), ((), ())),
        preferred_element_type=jnp.float32)


_PAD_BLK = 8192


def _tc_pad_labels(label_table_t):
    return pl.pallas_call(
        _tc_padT_body,
        grid=(pl.cdiv(NUM_EMB, _PAD_BLK),),
        in_specs=[
            pl.BlockSpec((NUM_CLASSES, _PAD_BLK), lambda i: (0, i)),
            pl.BlockSpec((NUM_CLASSES, LAB_PAD), lambda i: (0, 0)),
        ],
        out_specs=pl.BlockSpec((_PAD_BLK, LAB_PAD), lambda i: (i, 0)),
        out_shape=jax.ShapeDtypeStruct((NUM_EMB, LAB_PAD), jnp.float32),
    )(label_table_t, jnp.asarray(_EYE_NP))


def _tc_upsample_body(x_ref, kt_ref, o_ref):
    x = x_ref[...].astype(jnp.bfloat16)
    o_ref[...] = lax.dot_general(
        kt_ref[...], x, (((1,), (1,)), ((), ())),
        preferred_element_type=jnp.float32)


_TC_BLK = 2048
_CH_IN = 256
_CH_OUT = 1024


def _tc_upsample(gathered):
    return pl.pallas_call(
        _tc_upsample_body,
        grid=(3, BATCH // _TC_BLK),
        in_specs=[
            pl.BlockSpec((_TC_BLK, _CH_IN), lambda c, i: (i, c)),
            pl.BlockSpec((_CH_OUT, _CH_IN), lambda c, i: (0, 0)),
        ],
        out_specs=pl.BlockSpec((_CH_OUT, _TC_BLK), lambda c, i: (c, i)),
        out_shape=jax.ShapeDtypeStruct((OUT_DIM, BATCH), jnp.float32),
    )(gathered, jnp.asarray(_KT_NP))


@jax.jit
def kernel(indices, data_table, label_table):
    gathered = _sc_gather_data(indices, data_table)
    labp = _tc_pad_labels(label_table.T)
    labp, _ = lax.optimization_barrier((labp, gathered))
    labels_t = _tc_slice_labels(_sc_gather_labels(indices, labp))
    imgs_t = _tc_upsample(gathered)
    imgs = imgs_t.reshape(3, 32, 32, BATCH).transpose(3, 0, 1, 2)
    return imgs, labels_t.T
